# baseline (speedup 1.0000x reference)
import jax
import jax.numpy as jnp
from jax.experimental import pallas as pl
from jax.experimental.pallas import tpu as pltpu


def _pp_kernel(idx_ref, logits_ref, boxes_ref, scale_ref, pos_ref, out_ref):
    x = logits_ref[0]
    n_q = x.shape[0]

    rowmax = jnp.max(x, axis=1, keepdims=True)
    denom = jnp.sum(jnp.exp(x - rowmax), axis=1, keepdims=True)
    pos = pos_ref[0]
    masked = jnp.where(pos > 1e-6, x, -jnp.inf)
    mmax = jnp.max(masked, axis=1, keepdims=True)
    s_col = jnp.exp(mmax - rowmax) / denom
    s_row = jnp.transpose(s_col)

    iota_m = jax.lax.broadcasted_iota(jnp.int32, (n_q, n_q), 0)
    iota_n = jax.lax.broadcasted_iota(jnp.int32, (n_q, n_q), 1)
    gt = s_col > s_row
    eq = (s_col == s_row) & (iota_m < iota_n)
    rank_row = jnp.sum((gt | eq).astype(jnp.float32), axis=0, keepdims=True)

    onehot = (iota_m == rank_row.astype(jnp.int32)).astype(jnp.float32)

    bx = boxes_ref[0]
    cx, cy, w, h = bx[:, 0:1], bx[:, 1:2], bx[:, 2:3], bx[:, 3:4]
    xyxy = jnp.concatenate(
        [cx - 0.5 * w, cy - 0.5 * h, cx + 0.5 * w, cy + 0.5 * h], axis=1
    )
    xyxy = xyxy * scale_ref[0]

    out_ref[0] = jax.lax.dot(
        onehot, xyxy, precision=jax.lax.Precision.HIGHEST,
        preferred_element_type=jnp.float32,
    )


def kernel(pred_logits, pred_boxes, target_sizes, positive_map, phrase_batch_idx):
    B, N, L = pred_logits.shape
    P = positive_map.shape[0]

    img_h = target_sizes[:, 0]
    img_w = target_sizes[:, 1]
    scale = jnp.stack([img_w, img_h, img_w, img_h], axis=1).reshape(B, 1, 4)
    pos3 = positive_map.reshape(P, 1, L)

    grid_spec = pltpu.PrefetchScalarGridSpec(
        num_scalar_prefetch=1,
        grid=(P,),
        in_specs=[
            pl.BlockSpec((1, N, L), lambda p, idx: (idx[p], 0, 0)),
            pl.BlockSpec((1, N, 4), lambda p, idx: (idx[p], 0, 0)),
            pl.BlockSpec((1, 1, 4), lambda p, idx: (idx[p], 0, 0)),
            pl.BlockSpec((1, 1, L), lambda p, idx: (p, 0, 0)),
        ],
        out_specs=pl.BlockSpec((1, N, 4), lambda p, idx: (p, 0, 0)),
    )
    return pl.pallas_call(
        _pp_kernel,
        grid_spec=grid_spec,
        out_shape=jax.ShapeDtypeStruct((P, N, 4), jnp.float32),
    )(phrase_batch_idx, pred_logits, pred_boxes, scale, pos3)


# speedup vs baseline: 1.8584x; 1.8584x over previous
import jax
import jax.numpy as jnp
from jax.experimental import pallas as pl
from jax.experimental.pallas import tpu as pltpu


def _pp_kernel(idx_ref, logits_ref, boxes_ref, scale_ref, pos_ref, out_ref):
    x = logits_ref[0]
    n_q = x.shape[0]

    rowmax = jnp.max(x, axis=1, keepdims=True)
    denom = jnp.sum(jnp.exp(x - rowmax), axis=1, keepdims=True)
    pos = pos_ref[0]
    masked = jnp.where(pos > 1e-6, x, -jnp.inf)
    mmax = jnp.max(masked, axis=1, keepdims=True)
    s_col = jnp.exp(mmax - rowmax) / denom
    s_row = jnp.transpose(s_col)

    iota_m = jax.lax.broadcasted_iota(jnp.int32, (n_q, n_q), 0)
    iota_n = jax.lax.broadcasted_iota(jnp.int32, (n_q, n_q), 1)
    gt = s_col > s_row
    eq = (s_col == s_row) & (iota_m < iota_n)
    rank_row = jnp.sum((gt | eq).astype(jnp.float32), axis=0, keepdims=True)

    onehot = (iota_m == rank_row.astype(jnp.int32)).astype(jnp.bfloat16)

    bx = boxes_ref[0]
    cx, cy, w, h = bx[:, 0:1], bx[:, 1:2], bx[:, 2:3], bx[:, 3:4]
    xyxy = jnp.concatenate(
        [cx - 0.5 * w, cy - 0.5 * h, cx + 0.5 * w, cy + 0.5 * h], axis=1
    )
    xyxy = xyxy * scale_ref[0]

    hi = xyxy.astype(jnp.bfloat16)
    r1 = xyxy - hi.astype(jnp.float32)
    mid = r1.astype(jnp.bfloat16)
    lo = (r1 - mid.astype(jnp.float32)).astype(jnp.bfloat16)
    rhs = jnp.concatenate([hi, mid, lo], axis=1)
    y = jax.lax.dot(onehot, rhs, preferred_element_type=jnp.float32)
    out_ref[0] = (y[:, 0:4] + y[:, 4:8]) + y[:, 8:12]


def kernel(pred_logits, pred_boxes, target_sizes, positive_map, phrase_batch_idx):
    B, N, L = pred_logits.shape
    P = positive_map.shape[0]

    img_h = target_sizes[:, 0]
    img_w = target_sizes[:, 1]
    scale = jnp.stack([img_w, img_h, img_w, img_h], axis=1).reshape(B, 1, 4)
    pos3 = positive_map.reshape(P, 1, L)

    grid_spec = pltpu.PrefetchScalarGridSpec(
        num_scalar_prefetch=1,
        grid=(P,),
        in_specs=[
            pl.BlockSpec((1, N, L), lambda p, idx: (idx[p], 0, 0)),
            pl.BlockSpec((1, N, 4), lambda p, idx: (idx[p], 0, 0)),
            pl.BlockSpec((1, 1, 4), lambda p, idx: (idx[p], 0, 0)),
            pl.BlockSpec((1, 1, L), lambda p, idx: (p, 0, 0)),
        ],
        out_specs=pl.BlockSpec((1, N, 4), lambda p, idx: (p, 0, 0)),
    )
    return pl.pallas_call(
        _pp_kernel,
        grid_spec=grid_spec,
        out_shape=jax.ShapeDtypeStruct((P, N, 4), jnp.float32),
    )(phrase_batch_idx, pred_logits, pred_boxes, scale, pos3)
